# Initial kernel scaffold; baseline (speedup 1.0000x reference)
import jax
import jax.numpy as jnp
from jax.experimental import pallas as pl


def kernel(key, structure_coordinates, mask, residue_index, chain_index, Y, Y_t, Y_m, W_pos, W_edge, ln_g, ln_b, W_proj, b_proj):
    raise NotImplementedError("write your pallas kernel here")


# speedup vs baseline: 1.2071x; 1.2071x over previous
import functools

import numpy as np
import jax
import jax.numpy as jnp
from jax import lax
from jax.experimental import pallas as pl
from jax.experimental.pallas import tpu as pltpu
from jax.experimental.pallas import tpu_sc as plsc

L = 2048
K = 48
NPE = 16
EF = 128

RB = 128
NBLK = L // RB
CH = 128
NCH = L // CH

R_FEAT = 32
RK = R_FEAT * K
NFEAT = L // R_FEAT

_BIGF = float(np.inf)
_BIGI = 2**30

_PAIRS = [(1, 1), (0, 0), (2, 2), (3, 3), (4, 4), (1, 0), (1, 2), (1, 3),
          (1, 4), (0, 2), (0, 3), (0, 4), (4, 2), (4, 3), (3, 2), (0, 1),
          (2, 1), (3, 1), (4, 1), (2, 0), (3, 0), (4, 0), (2, 4), (3, 4),
          (2, 3)]
NP_ = len(_PAIRS)


def _sel_consts():
  s_a = np.zeros((16, 3 * NP_), np.float32)
  s_b = np.zeros((16, 3 * NP_), np.float32)
  s3 = np.zeros((3 * NP_, NP_), np.float32)
  for p, (a, b) in enumerate(_PAIRS):
    for c in range(3):
      s_a[3 * a + c, 3 * p + c] = 1.0
      s_b[3 * b + c, 3 * p + c] = 1.0
      s3[3 * p + c, p] = 1.0
  s_e = np.zeros((NP_, NP_ * 16), np.float32)
  for p in range(NP_):
    s_e[p, 16 * p:16 * p + 16] = 1.0
  r_exp = np.zeros((RK, R_FEAT), np.float32)
  for r in range(RK):
    r_exp[r, r // K] = 1.0
  return s_a, s_b, s3, s_e, r_exp


def _topk_body(ca_ref, cat_ref, sc_ref, ci_ref, eidx_ref, x_ref, d_ref):
  s = sc_ref[...]
  n = s[:, 0:3]
  ca_r = s[:, 3:6]
  c_r = s[:, 6:9]
  bv = ca_r - n
  cv = c_r - ca_r
  a0 = bv[:, 1:2] * cv[:, 2:3] - bv[:, 2:3] * cv[:, 1:2]
  a1 = bv[:, 2:3] * cv[:, 0:1] - bv[:, 0:1] * cv[:, 2:3]
  a2 = bv[:, 0:1] * cv[:, 1:2] - bv[:, 1:2] * cv[:, 0:1]
  av = jnp.concatenate([a0, a1, a2], axis=1)
  cb = -0.58273431 * av + 0.56802827 * bv - 0.54067466 * cv + ca_r
  x_ref[...] = jnp.concatenate([s, cb, ci_ref[...]], axis=1)

  for ch in range(NCH):
    col = ca_ref[pl.ds(ch * CH, CH), :]
    dx = col[:, 0:1] - cat_ref[0:1, :]
    dy = col[:, 1:2] - cat_ref[1:2, :]
    dz = col[:, 2:3] - cat_ref[2:3, :]
    d_ref[pl.ds(ch * CH, CH), :] = jnp.sqrt(dx * dx + dy * dy + dz * dz
                                            + 1e-06)

  def step(t, carry):
    lv, li = carry

    def scan_chunk(ch, mc):
      m, mi = mc
      v = d_ref[pl.ds(ch * CH, CH), :]
      io = lax.broadcasted_iota(jnp.int32, (CH, RB), 0) + ch * CH
      cand = (v > lv) | ((v == lv) & (io > li))
      vm = jnp.where(cand, v, _BIGF)
      m_c = jnp.min(vm, axis=0, keepdims=True)
      im_c = jnp.min(jnp.where(vm == m_c, io, _BIGI), axis=0, keepdims=True)
      take = (m_c < m) | ((m_c == m) & (im_c < mi))
      return jnp.where(take, m_c, m), jnp.where(take, im_c, mi)

    m0 = jnp.full((1, RB), _BIGF, jnp.float32)
    i0 = jnp.full((1, RB), _BIGI, jnp.int32)
    m, mi = lax.fori_loop(0, NCH, scan_chunk, (m0, i0))
    eidx_ref[pl.ds(t, 1), :] = mi
    return m, mi

  lv0 = jnp.full((1, RB), -_BIGF, jnp.float32)
  li0 = jnp.full((1, RB), -1, jnp.int32)
  lax.fori_loop(0, K, step, (lv0, li0))


def _topk_call(ca, ca_t, sc2, ci_f, interpret=False):
  return pl.pallas_call(
      _topk_body,
      grid=(NBLK,),
      in_specs=[
          pl.BlockSpec((L, 3), lambda b: (0, 0)),
          pl.BlockSpec((8, RB), lambda b: (0, b)),
          pl.BlockSpec((RB, 12), lambda b: (b, 0)),
          pl.BlockSpec((RB, 1), lambda b: (b, 0)),
      ],
      out_specs=[
          pl.BlockSpec((K, RB), lambda b: (0, b)),
          pl.BlockSpec((RB, 16), lambda b: (b, 0)),
      ],
      out_shape=[
          jax.ShapeDtypeStruct((K, L), jnp.int32),
          jax.ShapeDtypeStruct((L, 16), jnp.float32),
      ],
      scratch_shapes=[pltpu.VMEM((L, RB), jnp.float32)],
      interpret=interpret,
  )(ca, ca_t, sc2, ci_f)


_GB_PER_W = (L * K) // 32
_GCHUNKS = _GB_PER_W // 128


def _sc_gather(idx3, table):
  info = plsc.get_sparse_core_info()
  nc = info.num_cores

  @functools.partial(
      pl.kernel,
      mesh=plsc.VectorSubcoreMesh(core_axis_name="c", subcore_axis_name="s"),
      compiler_params=pltpu.CompilerParams(use_tc_tiling_on_sc=False),
      out_type=jax.ShapeDtypeStruct((L * K, 16), jnp.float32),
      scratch_types=[
          pltpu.VMEM((_GCHUNKS, 128), jnp.int32),
          pltpu.VMEM((_GB_PER_W, 16), jnp.float32),
          pltpu.SemaphoreType.DMA,
      ],
  )
  def gath(idx_hbm, table_hbm, out_hbm, idx_v, rows_v, sem):
    wid = lax.axis_index("s") * nc + lax.axis_index("c")
    pltpu.sync_copy(idx_hbm.at[wid], idx_v)

    def chunk(c, carry):
      pltpu.async_copy(table_hbm.at[idx_v.at[c]],
                       rows_v.at[pl.ds(c * 128, 128), :], sem).wait()
      return carry

    lax.fori_loop(0, _GCHUNKS, chunk, 0)
    pltpu.sync_copy(rows_v, out_hbm.at[pl.ds(wid * _GB_PER_W, _GB_PER_W), :])

  return gath(idx3, table)


def _feat_body(x_ref, xn_ref, ii_ref, jj_ref, sa_ref, sb_ref, s3_ref,
               se_ref, rexp_ref, mu_ref, wc_ref, we2_ref, lng_ref, lnb_ref,
               wp_ref, bp_ref, out_ref):
  hi = jax.lax.Precision.HIGHEST
  xe = jnp.dot(rexp_ref[...], x_ref[...], precision=hi,
               preferred_element_type=jnp.float32)
  xn = xn_ref[...]
  a = jnp.dot(xe, sa_ref[...], precision=hi,
              preferred_element_type=jnp.float32)
  b = jnp.dot(xn, sb_ref[...], precision=hi,
              preferred_element_type=jnp.float32)
  sq = (a - b) ** 2
  d2 = jnp.dot(sq, s3_ref[...], precision=hi,
               preferred_element_type=jnp.float32)
  d = jnp.sqrt(d2 + 1e-06)
  x400 = jnp.dot(d, se_ref[...], precision=hi,
                 preferred_element_type=jnp.float32)
  rbf = jnp.exp(-(((x400 - mu_ref[...]) * 0.8) ** 2))

  d_off = jnp.clip(ii_ref[...] - jj_ref[...] + NPE, 0, 2 * NPE)
  oh = (lax.broadcasted_iota(jnp.int32, (RK, 2 * NPE + 1), 1)
        == d_off).astype(jnp.float32)
  chain = 1.0 - (xe[:, 15:16] == xn[:, 15:16]).astype(jnp.float32)
  dcomb = jnp.concatenate([oh, chain], axis=1)

  e = (jnp.dot(dcomb, wc_ref[...], precision=hi,
               preferred_element_type=jnp.float32)
       + jnp.dot(rbf, we2_ref[...], precision=hi,
                 preferred_element_type=jnp.float32))
  mu = jnp.mean(e, axis=1, keepdims=True)
  var = jnp.mean((e - mu) ** 2, axis=1, keepdims=True)
  e = (e - mu) / jnp.sqrt(var + 1e-05) * lng_ref[...] + lnb_ref[...]
  out_ref[...] = jnp.dot(e, wp_ref[...], precision=hi,
                         preferred_element_type=jnp.float32) + bp_ref[...]


def _feat_call(x, xn, ii, jj, sa, sb, s3, se, rexp, mu400, wct, we2t,
               lng, lnb, wpt, bp, interpret=False):
  full = lambda shape: pl.BlockSpec(shape, lambda b: (0, 0))
  return pl.pallas_call(
      _feat_body,
      grid=(NFEAT,),
      in_specs=[
          pl.BlockSpec((R_FEAT, 16), lambda b: (b, 0)),
          pl.BlockSpec((RK, 16), lambda b: (b, 0)),
          pl.BlockSpec((RK, 1), lambda b: (b, 0)),
          pl.BlockSpec((RK, 1), lambda b: (b, 0)),
          full((16, 3 * NP_)),
          full((16, 3 * NP_)),
          full((3 * NP_, NP_)),
          full((NP_, 400)),
          full((RK, R_FEAT)),
          full((1, 400)),
          full((2 * NPE + 2, EF)),
          full((400, EF)),
          full((1, EF)),
          full((1, EF)),
          full((EF, EF)),
          full((1, EF)),
      ],
      out_specs=pl.BlockSpec((RK, EF), lambda b: (b, 0)),
      out_shape=jax.ShapeDtypeStruct((L * K, EF), jnp.float32),
      interpret=interpret,
  )(x, xn, ii, jj, sa, sb, s3, se, rexp, mu400, wct, we2t, lng, lnb, wpt, bp)


def kernel(key, structure_coordinates, mask, residue_index, chain_index,
           Y, Y_t, Y_m, W_pos, W_edge, ln_g, ln_b, W_proj, b_proj):
  sc2 = structure_coordinates.reshape(L, 12)
  ci_f = chain_index.astype(jnp.float32).reshape(L, 1)
  ca = structure_coordinates[:, 1, :]
  ca_t = jnp.concatenate([ca.T, jnp.zeros((5, L), jnp.float32)], axis=0)

  eidx_t, x = _topk_call(ca, ca_t, sc2, ci_f)
  e_idx = eidx_t.T

  jflat = e_idx.reshape(L * K)
  xn = _sc_gather(jflat.reshape(32, _GCHUNKS, 128), x)

  sa, sb, s3, se, rexp = _sel_consts()
  mu400 = jnp.asarray(np.tile(np.linspace(2.0, 22.0, 16,
                                          dtype=np.float32), NP_)[None, :])
  wct = (W_edge[:, :16] @ W_pos).T
  we2t = W_edge[:, 16:].T
  ii = jnp.repeat(jnp.arange(L, dtype=jnp.int32), K).reshape(L * K, 1)
  jj = jflat.reshape(L * K, 1).astype(jnp.int32)

  e = _feat_call(x, xn, ii, jj,
                 jnp.asarray(sa), jnp.asarray(sb), jnp.asarray(s3),
                 jnp.asarray(se), jnp.asarray(rexp), mu400, wct, we2t,
                 ln_g.reshape(1, EF), ln_b.reshape(1, EF), W_proj.T,
                 b_proj.reshape(1, EF))
  return e.reshape(L, K, EF), e_idx


# speedup vs baseline: 1.8244x; 1.5114x over previous
import functools

import numpy as np
import jax
import jax.numpy as jnp
from jax import lax
from jax.experimental import pallas as pl
from jax.experimental.pallas import tpu as pltpu
from jax.experimental.pallas import tpu_sc as plsc

L = 2048
K = 48
NPE = 16
EF = 128

RB = 128
NBLK = L // RB
CH = 128
NCH = L // CH

R_FEAT = 32
RK = R_FEAT * K
NFEAT = L // R_FEAT

_BIGF = float(np.inf)
_BIGI = 2**30

_PAIRS = [(1, 1), (0, 0), (2, 2), (3, 3), (4, 4), (1, 0), (1, 2), (1, 3),
          (1, 4), (0, 2), (0, 3), (0, 4), (4, 2), (4, 3), (3, 2), (0, 1),
          (2, 1), (3, 1), (4, 1), (2, 0), (3, 0), (4, 0), (2, 4), (3, 4),
          (2, 3)]
NP_ = len(_PAIRS)


def _sel_consts():
  sa = np.zeros((16, 80), np.float32)
  sb = np.zeros((16, 80), np.float32)
  for p, (a, b) in enumerate(_PAIRS):
    for c in range(3):
      sa[3 * a + c, 3 * p + c] = 1.0
      sb[3 * b + c, 3 * p + c] = 1.0
  sa[15, 79] = 1.0
  sb[15, 79] = 1.0
  s3e = np.zeros((80, 400), np.float32)
  for p in range(NP_):
    for c in range(3):
      s3e[3 * p + c, 16 * p:16 * p + 16] = 1.0
  r_exp = np.zeros((RK, R_FEAT), np.float32)
  for r in range(RK):
    r_exp[r, r // K] = 1.0
  return sa, sb, s3e, r_exp


def _topk_body(ca_ref, cat_ref, sc_ref, ci_ref, eidx_ref, x_ref, d_ref):
  s = sc_ref[...]
  n = s[:, 0:3]
  ca_r = s[:, 3:6]
  c_r = s[:, 6:9]
  bv = ca_r - n
  cv = c_r - ca_r
  a0 = bv[:, 1:2] * cv[:, 2:3] - bv[:, 2:3] * cv[:, 1:2]
  a1 = bv[:, 2:3] * cv[:, 0:1] - bv[:, 0:1] * cv[:, 2:3]
  a2 = bv[:, 0:1] * cv[:, 1:2] - bv[:, 1:2] * cv[:, 0:1]
  av = jnp.concatenate([a0, a1, a2], axis=1)
  cb = -0.58273431 * av + 0.56802827 * bv - 0.54067466 * cv + ca_r
  x_ref[...] = jnp.concatenate([s, cb, ci_ref[...]], axis=1)

  for ch in range(NCH):
    col = ca_ref[pl.ds(ch * CH, CH), :]
    dx = col[:, 0:1] - cat_ref[0:1, :]
    dy = col[:, 1:2] - cat_ref[1:2, :]
    dz = col[:, 2:3] - cat_ref[2:3, :]
    d_ref[pl.ds(ch * CH, CH), :] = jnp.sqrt(dx * dx + dy * dy + dz * dz
                                            + 1e-06)

  def step(t, li):

    def scan_chunk(ch, mc):
      m, mi = mc
      io = lax.broadcasted_iota(jnp.int32, (CH, RB), 0) + ch * CH
      v = jnp.where(io == li, _BIGF, d_ref[pl.ds(ch * CH, CH), :])
      d_ref[pl.ds(ch * CH, CH), :] = v
      m_c = jnp.min(v, axis=0, keepdims=True)
      im_c = jnp.min(jnp.where(v == m_c, io, _BIGI), axis=0, keepdims=True)
      take = (m_c < m) | ((m_c == m) & (im_c < mi))
      return jnp.where(take, m_c, m), jnp.where(take, im_c, mi)

    m0 = jnp.full((1, RB), _BIGF, jnp.float32)
    i0 = jnp.full((1, RB), _BIGI, jnp.int32)
    _, mi = lax.fori_loop(0, NCH, scan_chunk, (m0, i0))
    eidx_ref[pl.ds(t, 1), :] = mi
    return mi

  lax.fori_loop(0, K, step, jnp.full((1, RB), -1, jnp.int32))


def _topk_call(ca, ca_t, sc2, ci_f, interpret=False):
  return pl.pallas_call(
      _topk_body,
      grid=(NBLK,),
      in_specs=[
          pl.BlockSpec((L, 3), lambda b: (0, 0)),
          pl.BlockSpec((8, RB), lambda b: (0, b)),
          pl.BlockSpec((RB, 12), lambda b: (b, 0)),
          pl.BlockSpec((RB, 1), lambda b: (b, 0)),
      ],
      out_specs=[
          pl.BlockSpec((K, RB), lambda b: (0, b)),
          pl.BlockSpec((RB, 16), lambda b: (b, 0)),
      ],
      out_shape=[
          jax.ShapeDtypeStruct((K, L), jnp.int32),
          jax.ShapeDtypeStruct((L, 16), jnp.float32),
      ],
      scratch_shapes=[pltpu.VMEM((L, RB), jnp.float32)],
      interpret=interpret,
  )(ca, ca_t, sc2, ci_f)


_GB_PER_W = (L * K) // 32
_GCHUNKS = _GB_PER_W // 128


def _sc_gather(idx3, table):
  info = plsc.get_sparse_core_info()
  nc = info.num_cores

  @functools.partial(
      pl.kernel,
      mesh=plsc.VectorSubcoreMesh(core_axis_name="c", subcore_axis_name="s"),
      compiler_params=pltpu.CompilerParams(use_tc_tiling_on_sc=False),
      out_type=jax.ShapeDtypeStruct((L * K, 16), jnp.float32),
      scratch_types=[
          pltpu.VMEM((_GCHUNKS, 128), jnp.int32),
          pltpu.VMEM((_GB_PER_W, 16), jnp.float32),
          pltpu.SemaphoreType.DMA,
      ],
  )
  def gath(idx_hbm, table_hbm, out_hbm, idx_v, rows_v, sem):
    wid = lax.axis_index("s") * nc + lax.axis_index("c")
    pltpu.sync_copy(idx_hbm.at[wid], idx_v)

    def chunk(c, carry):
      pltpu.async_copy(table_hbm.at[idx_v.at[c]],
                       rows_v.at[pl.ds(c * 128, 128), :], sem).wait()
      return carry

    lax.fori_loop(0, _GCHUNKS, chunk, 0)
    pltpu.sync_copy(rows_v, out_hbm.at[pl.ds(wid * _GB_PER_W, _GB_PER_W), :])

  return gath(idx3, table)


def _feat_body(x_ref, xn_ref, ii_ref, jj_ref, sa_ref, sb_ref, s3e_ref,
               rexp_ref, mu_ref, wbig_ref, lng_ref, lnb_ref,
               wp_ref, bp_ref, out_ref):
  dot = functools.partial(jnp.dot, precision=jax.lax.Precision.HIGHEST,
                          preferred_element_type=jnp.float32)
  dot_lo = functools.partial(jnp.dot, precision=jax.lax.Precision.DEFAULT,
                             preferred_element_type=jnp.float32)
  xa = dot(x_ref[...], sa_ref[...])
  w_top = jnp.concatenate([xa, jnp.zeros((R_FEAT, 176), jnp.float32)], 1)
  w_bot = jnp.concatenate([jnp.zeros((16, 128), jnp.float32), sb_ref[...],
                           jnp.zeros((16, 48), jnp.float32)], 1)
  wsel = jnp.concatenate([w_top, w_bot], 0)
  inp = jnp.concatenate([rexp_ref[...], xn_ref[...]], 1)
  m1 = dot(inp, wsel)
  sq = (m1[:, 0:80] - m1[:, 128:208]) ** 2
  x4 = dot(sq, s3e_ref[...])
  d4 = jnp.sqrt(x4 + 1e-06)
  rbf = jnp.exp(-(((d4 - mu_ref[...]) * 0.8) ** 2))

  d_off = jnp.clip(ii_ref[...] - jj_ref[...] + NPE, 0, 2 * NPE)
  oh = (lax.broadcasted_iota(jnp.int32, (RK, 2 * NPE + 1), 1)
        == d_off).astype(jnp.float32)
  chain = 1.0 - (m1[:, 79:80] == m1[:, 207:208]).astype(jnp.float32)
  feat = jnp.concatenate([rbf, oh, chain], axis=1)

  e = dot_lo(feat, wbig_ref[...])
  mu = jnp.mean(e, axis=1, keepdims=True)
  var = jnp.mean((e - mu) ** 2, axis=1, keepdims=True)
  e = (e - mu) / jnp.sqrt(var + 1e-05) * lng_ref[...] + lnb_ref[...]
  out_ref[...] = dot_lo(e, wp_ref[...]) + bp_ref[...]


def _feat_call(x, xn, ii, jj, sa, sb, s3e, rexp, mu400, wbig,
               lng, lnb, wpt, bp, interpret=False):
  full = lambda shape: pl.BlockSpec(shape, lambda b: (0, 0))
  return pl.pallas_call(
      _feat_body,
      grid=(NFEAT,),
      in_specs=[
          pl.BlockSpec((R_FEAT, 16), lambda b: (b, 0)),
          pl.BlockSpec((RK, 16), lambda b: (b, 0)),
          pl.BlockSpec((RK, 1), lambda b: (b, 0)),
          pl.BlockSpec((RK, 1), lambda b: (b, 0)),
          full((16, 80)),
          full((16, 80)),
          full((80, 400)),
          full((RK, R_FEAT)),
          full((1, 400)),
          full((434, EF)),
          full((1, EF)),
          full((1, EF)),
          full((EF, EF)),
          full((1, EF)),
      ],
      out_specs=pl.BlockSpec((RK, EF), lambda b: (b, 0)),
      out_shape=jax.ShapeDtypeStruct((L * K, EF), jnp.float32),
      interpret=interpret,
  )(x, xn, ii, jj, sa, sb, s3e, rexp, mu400, wbig, lng, lnb, wpt, bp)


def kernel(key, structure_coordinates, mask, residue_index, chain_index,
           Y, Y_t, Y_m, W_pos, W_edge, ln_g, ln_b, W_proj, b_proj):
  sc2 = structure_coordinates.reshape(L, 12)
  ci_f = chain_index.astype(jnp.float32).reshape(L, 1)
  ca = structure_coordinates[:, 1, :]
  ca_t = jnp.concatenate([ca.T, jnp.zeros((5, L), jnp.float32)], axis=0)

  eidx_t, x = _topk_call(ca, ca_t, sc2, ci_f)
  e_idx = eidx_t.T

  jflat = e_idx.reshape(L * K)
  xn = _sc_gather(jflat.reshape(32, _GCHUNKS, 128), x)

  sa, sb, s3e, rexp = _sel_consts()
  mu400 = jnp.asarray(np.tile(np.linspace(2.0, 22.0, 16,
                                          dtype=np.float32), NP_)[None, :])
  wbig = jnp.concatenate([W_edge[:, 16:].T,
                          (W_edge[:, :16] @ W_pos).T], axis=0)
  ii = jnp.repeat(jnp.arange(L, dtype=jnp.int32), K).reshape(L * K, 1)
  jj = jflat.reshape(L * K, 1).astype(jnp.int32)

  e = _feat_call(x, xn, ii, jj,
                 jnp.asarray(sa), jnp.asarray(sb), jnp.asarray(s3e),
                 jnp.asarray(rexp), mu400, wbig,
                 ln_g.reshape(1, EF), ln_b.reshape(1, EF), W_proj.T,
                 b_proj.reshape(1, EF))
  return e.reshape(L, K, EF), e_idx


# speedup vs baseline: 2.8094x; 1.5399x over previous
import functools

import numpy as np
import jax
import jax.numpy as jnp
from jax import lax
from jax.experimental import pallas as pl
from jax.experimental.pallas import tpu as pltpu
from jax.experimental.pallas import tpu_sc as plsc

L = 2048
K = 48
NPE = 16
EF = 128

RB = 128
NBLK = L // RB
CH = 128
NCH = L // CH

R_FEAT = 32
RK = R_FEAT * K
NFEAT = L // R_FEAT

_BIGF = float(np.inf)
_BIGI = 2**30

_PAIRS = [(1, 1), (0, 0), (2, 2), (3, 3), (4, 4), (1, 0), (1, 2), (1, 3),
          (1, 4), (0, 2), (0, 3), (0, 4), (4, 2), (4, 3), (3, 2), (0, 1),
          (2, 1), (3, 1), (4, 1), (2, 0), (3, 0), (4, 0), (2, 4), (3, 4),
          (2, 3)]
NP_ = len(_PAIRS)


def _sel_consts():
  sa = np.zeros((16, 80), np.float32)
  sb = np.zeros((16, 80), np.float32)
  for p, (a, b) in enumerate(_PAIRS):
    for c in range(3):
      sa[3 * a + c, 3 * p + c] = 1.0
      sb[3 * b + c, 3 * p + c] = 1.0
  sa[15, 79] = 1.0
  sb[15, 79] = 1.0
  s3e = np.zeros((80, 400), np.float32)
  for p in range(NP_):
    for c in range(3):
      s3e[3 * p + c, 16 * p:16 * p + 16] = 1.0
  r_exp = np.zeros((RK, R_FEAT), np.float32)
  for r in range(RK):
    r_exp[r, r // K] = 1.0
  return sa, sb, s3e, r_exp


def _topk_body(ca_ref, cat_ref, sc_ref, ci_ref, eidx_ref, x_ref, d_ref):
  s = sc_ref[...]
  n = s[:, 0:3]
  ca_r = s[:, 3:6]
  c_r = s[:, 6:9]
  bv = ca_r - n
  cv = c_r - ca_r
  a0 = bv[:, 1:2] * cv[:, 2:3] - bv[:, 2:3] * cv[:, 1:2]
  a1 = bv[:, 2:3] * cv[:, 0:1] - bv[:, 0:1] * cv[:, 2:3]
  a2 = bv[:, 0:1] * cv[:, 1:2] - bv[:, 1:2] * cv[:, 0:1]
  av = jnp.concatenate([a0, a1, a2], axis=1)
  cb = -0.58273431 * av + 0.56802827 * bv - 0.54067466 * cv + ca_r
  x_ref[...] = jnp.concatenate([s, cb, ci_ref[...]], axis=1)

  for ch in range(NCH):
    col = ca_ref[pl.ds(ch * CH, CH), :]
    dx = col[:, 0:1] - cat_ref[0:1, :]
    dy = col[:, 1:2] - cat_ref[1:2, :]
    dz = col[:, 2:3] - cat_ref[2:3, :]
    d_ref[pl.ds(ch * CH, CH), :] = jnp.sqrt(dx * dx + dy * dy + dz * dz
                                            + 1e-06)

  def step(t, li):

    def scan_chunk(ch, mc):
      rv, ri = mc
      io = lax.broadcasted_iota(jnp.int32, (CH, RB), 0) + ch * CH
      v = jnp.where(io == li, _BIGF, d_ref[pl.ds(ch * CH, CH), :])
      d_ref[pl.ds(ch * CH, CH), :] = v
      take = v < rv
      return jnp.where(take, v, rv), jnp.where(take, io, ri)

    v0 = jnp.full((CH, RB), _BIGF, jnp.float32)
    i0 = jnp.full((CH, RB), _BIGI, jnp.int32)
    rv, ri = lax.fori_loop(0, NCH, scan_chunk, (v0, i0))
    m = jnp.min(rv, axis=0, keepdims=True)
    mi = jnp.min(jnp.where(rv == m, ri, _BIGI), axis=0, keepdims=True)
    eidx_ref[pl.ds(t, 1), :] = mi
    return mi

  lax.fori_loop(0, K, step, jnp.full((1, RB), -1, jnp.int32))


def _topk_call(ca, ca_t, sc2, ci_f, interpret=False):
  return pl.pallas_call(
      _topk_body,
      grid=(NBLK,),
      in_specs=[
          pl.BlockSpec((L, 3), lambda b: (0, 0)),
          pl.BlockSpec((8, RB), lambda b: (0, b)),
          pl.BlockSpec((RB, 12), lambda b: (b, 0)),
          pl.BlockSpec((RB, 1), lambda b: (b, 0)),
      ],
      out_specs=[
          pl.BlockSpec((K, RB), lambda b: (0, b)),
          pl.BlockSpec((RB, 16), lambda b: (b, 0)),
      ],
      out_shape=[
          jax.ShapeDtypeStruct((K, L), jnp.int32),
          jax.ShapeDtypeStruct((L, 16), jnp.float32),
      ],
      scratch_shapes=[pltpu.VMEM((L, RB), jnp.float32)],
      interpret=interpret,
  )(ca, ca_t, sc2, ci_f)


_GB_PER_W = (L * K) // 32
_GCHUNKS = _GB_PER_W // 128


def _sc_gather(idx3, table):
  info = plsc.get_sparse_core_info()
  nc = info.num_cores

  @functools.partial(
      pl.kernel,
      mesh=plsc.VectorSubcoreMesh(core_axis_name="c", subcore_axis_name="s"),
      compiler_params=pltpu.CompilerParams(use_tc_tiling_on_sc=False),
      out_type=jax.ShapeDtypeStruct((L * K, 16), jnp.float32),
      scratch_types=[
          pltpu.VMEM((_GCHUNKS, 128), jnp.int32),
          pltpu.VMEM((_GB_PER_W, 16), jnp.float32),
          pltpu.SemaphoreType.DMA,
      ],
  )
  def gath(idx_hbm, table_hbm, out_hbm, idx_v, rows_v, sem):
    wid = lax.axis_index("s") * nc + lax.axis_index("c")
    pltpu.sync_copy(idx_hbm.at[wid], idx_v)

    def chunk(c, carry):
      pltpu.async_copy(table_hbm.at[idx_v.at[c]],
                       rows_v.at[pl.ds(c * 128, 128), :], sem).wait()
      return carry

    lax.fori_loop(0, _GCHUNKS, chunk, 0)
    pltpu.sync_copy(rows_v, out_hbm.at[pl.ds(wid * _GB_PER_W, _GB_PER_W), :])

  return gath(idx3, table)


def _feat_body(x_ref, xn_ref, ii_ref, jj_ref, sa_ref, sb_ref, s3e_ref,
               rexp_ref, mu_ref, wbig_ref, lng_ref, lnb_ref,
               wp_ref, bp_ref, out_ref):
  dot = functools.partial(jnp.dot, precision=jax.lax.Precision.HIGHEST,
                          preferred_element_type=jnp.float32)
  dot_lo = functools.partial(jnp.dot, precision=jax.lax.Precision.DEFAULT,
                             preferred_element_type=jnp.float32)

  def split_dot(vals, sel):
    hi_p = vals.astype(jnp.bfloat16).astype(jnp.float32)
    return dot_lo(hi_p, sel) + dot_lo(vals - hi_p, sel)

  xa = dot(x_ref[...], sa_ref[...])
  xa_hi = xa.astype(jnp.bfloat16).astype(jnp.float32)
  a = (dot_lo(rexp_ref[...], xa_hi)
       + dot_lo(rexp_ref[...], xa - xa_hi))
  xn = xn_ref[...]
  xn_hi = xn.astype(jnp.bfloat16).astype(jnp.float32)
  b = (dot_lo(xn_hi, sb_ref[...])
       + dot_lo(xn - xn_hi, sb_ref[...]))
  sq = (a - b) ** 2
  x4 = split_dot(sq, s3e_ref[...])
  d4 = jnp.sqrt(x4 + 1e-06)
  rbf = jnp.exp(-(((d4 - mu_ref[...]) * 0.8) ** 2))

  d_off = jnp.clip(ii_ref[...] - jj_ref[...] + NPE, 0, 2 * NPE)
  oh = (lax.broadcasted_iota(jnp.int32, (RK, 2 * NPE + 1), 1)
        == d_off).astype(jnp.float32)
  chain = 1.0 - (a[:, 79:80] == b[:, 79:80]).astype(jnp.float32)
  feat = jnp.concatenate([rbf, oh, chain], axis=1)

  e = dot_lo(feat, wbig_ref[...])
  mu = jnp.mean(e, axis=1, keepdims=True)
  var = jnp.mean((e - mu) ** 2, axis=1, keepdims=True)
  e = (e - mu) / jnp.sqrt(var + 1e-05) * lng_ref[...] + lnb_ref[...]
  out_ref[...] = dot_lo(e, wp_ref[...]) + bp_ref[...]


def _feat_call(x, xn, ii, jj, sa, sb, s3e, rexp, mu400, wbig,
               lng, lnb, wpt, bp, interpret=False):
  full = lambda shape: pl.BlockSpec(shape, lambda b: (0, 0))
  return pl.pallas_call(
      _feat_body,
      grid=(NFEAT,),
      in_specs=[
          pl.BlockSpec((R_FEAT, 16), lambda b: (b, 0)),
          pl.BlockSpec((RK, 16), lambda b: (b, 0)),
          pl.BlockSpec((RK, 1), lambda b: (b, 0)),
          pl.BlockSpec((RK, 1), lambda b: (b, 0)),
          full((16, 80)),
          full((16, 80)),
          full((80, 400)),
          full((RK, R_FEAT)),
          full((1, 400)),
          full((434, EF)),
          full((1, EF)),
          full((1, EF)),
          full((EF, EF)),
          full((1, EF)),
      ],
      out_specs=pl.BlockSpec((RK, EF), lambda b: (b, 0)),
      out_shape=jax.ShapeDtypeStruct((L * K, EF), jnp.float32),
      interpret=interpret,
  )(x, xn, ii, jj, sa, sb, s3e, rexp, mu400, wbig, lng, lnb, wpt, bp)


def kernel(key, structure_coordinates, mask, residue_index, chain_index,
           Y, Y_t, Y_m, W_pos, W_edge, ln_g, ln_b, W_proj, b_proj):
  sc2 = structure_coordinates.reshape(L, 12)
  ci_f = chain_index.astype(jnp.float32).reshape(L, 1)
  ca = structure_coordinates[:, 1, :]
  ca_t = jnp.concatenate([ca.T, jnp.zeros((5, L), jnp.float32)], axis=0)

  eidx_t, x = _topk_call(ca, ca_t, sc2, ci_f)
  e_idx = eidx_t.T

  jflat = e_idx.reshape(L * K)
  xn = _sc_gather(jflat.reshape(32, _GCHUNKS, 128), x)

  sa, sb, s3e, rexp = _sel_consts()
  mu400 = jnp.asarray(np.tile(np.linspace(2.0, 22.0, 16,
                                          dtype=np.float32), NP_)[None, :])
  wbig = jnp.concatenate([W_edge[:, 16:].T,
                          (W_edge[:, :16] @ W_pos).T], axis=0)
  ii = jnp.repeat(jnp.arange(L, dtype=jnp.int32), K).reshape(L * K, 1)
  jj = jflat.reshape(L * K, 1).astype(jnp.int32)

  e = _feat_call(x, xn, ii, jj,
                 jnp.asarray(sa), jnp.asarray(sb), jnp.asarray(s3e),
                 jnp.asarray(rexp), mu400, wbig,
                 ln_g.reshape(1, EF), ln_b.reshape(1, EF), W_proj.T,
                 b_proj.reshape(1, EF))
  return e.reshape(L, K, EF), e_idx


# speedup vs baseline: 3.4402x; 1.2245x over previous
import functools

import numpy as np
import jax
import jax.numpy as jnp
from jax import lax
from jax.experimental import pallas as pl
from jax.experimental.pallas import tpu as pltpu
from jax.experimental.pallas import tpu_sc as plsc

L = 2048
K = 48
NPE = 16
EF = 128

RB = 128
NBLK = L // RB
CH = 128
NCH = L // CH

R_FEAT = 32
RK = R_FEAT * K
NFEAT = L // R_FEAT

_BIGF = float(np.inf)
_BIGI = 2**30

_PAIRS = [(1, 1), (0, 0), (2, 2), (3, 3), (4, 4), (1, 0), (1, 2), (1, 3),
          (1, 4), (0, 2), (0, 3), (0, 4), (4, 2), (4, 3), (3, 2), (0, 1),
          (2, 1), (3, 1), (4, 1), (2, 0), (3, 0), (4, 0), (2, 4), (3, 4),
          (2, 3)]
NP_ = len(_PAIRS)


def _sel_consts():
  sa = np.zeros((16, 80), np.float32)
  sb = np.zeros((16, 80), np.float32)
  for p, (a, b) in enumerate(_PAIRS):
    for c in range(3):
      sa[3 * a + c, 3 * p + c] = 1.0
      sb[3 * b + c, 3 * p + c] = 1.0
  sa[15, 79] = 1.0
  sb[15, 79] = 1.0
  s3e = np.zeros((80, 400), np.float32)
  for p in range(NP_):
    for c in range(3):
      s3e[3 * p + c, 16 * p:16 * p + 16] = 1.0
  r_exp = np.zeros((RK, R_FEAT), np.float32)
  for r in range(RK):
    r_exp[r, r // K] = 1.0
  return sa, sb, s3e, r_exp


def _topk_body(ca_ref, cat_ref, sc_ref, ci_ref, eidx_ref, x_ref, d_ref,
               io_ref):
  s = sc_ref[...]
  n = s[:, 0:3]
  ca_r = s[:, 3:6]
  c_r = s[:, 6:9]
  bv = ca_r - n
  cv = c_r - ca_r
  a0 = bv[:, 1:2] * cv[:, 2:3] - bv[:, 2:3] * cv[:, 1:2]
  a1 = bv[:, 2:3] * cv[:, 0:1] - bv[:, 0:1] * cv[:, 2:3]
  a2 = bv[:, 0:1] * cv[:, 1:2] - bv[:, 1:2] * cv[:, 0:1]
  av = jnp.concatenate([a0, a1, a2], axis=1)
  cb = -0.58273431 * av + 0.56802827 * bv - 0.54067466 * cv + ca_r
  x_ref[...] = jnp.concatenate([s, cb, ci_ref[...]], axis=1)

  for ch in range(NCH):
    col = ca_ref[pl.ds(ch * CH, CH), :]
    dx = col[:, 0:1] - cat_ref[0:1, :]
    dy = col[:, 1:2] - cat_ref[1:2, :]
    dz = col[:, 2:3] - cat_ref[2:3, :]
    d_ref[pl.ds(ch * CH, CH), :] = jnp.sqrt(dx * dx + dy * dy + dz * dz
                                            + 1e-06)
    io_ref[pl.ds(ch * CH, CH), :] = (
        lax.broadcasted_iota(jnp.int32, (CH, RB), 0) + ch * CH)

  def step(t, li):

    def scan_chunk(ch, mc):
      rv, ri = mc
      io = io_ref[pl.ds(ch * CH, CH), :]
      v = jnp.where(io == li, _BIGF, d_ref[pl.ds(ch * CH, CH), :])
      d_ref[pl.ds(ch * CH, CH), :] = v
      take = v < rv
      return jnp.where(take, v, rv), jnp.where(take, io, ri)

    v0 = jnp.full((CH, RB), _BIGF, jnp.float32)
    i0 = jnp.full((CH, RB), _BIGI, jnp.int32)
    rv, ri = lax.fori_loop(0, NCH, scan_chunk, (v0, i0), unroll=2)
    m = jnp.min(rv, axis=0, keepdims=True)
    mi = jnp.min(jnp.where(rv == m, ri, _BIGI), axis=0, keepdims=True)
    eidx_ref[pl.ds(t, 1), :] = mi
    return mi

  lax.fori_loop(0, K, step, jnp.full((1, RB), -1, jnp.int32))


def _topk_call(ca, ca_t, sc2, ci_f, interpret=False):
  return pl.pallas_call(
      _topk_body,
      grid=(NBLK,),
      in_specs=[
          pl.BlockSpec((L, 3), lambda b: (0, 0)),
          pl.BlockSpec((8, RB), lambda b: (0, b)),
          pl.BlockSpec((RB, 12), lambda b: (b, 0)),
          pl.BlockSpec((RB, 1), lambda b: (b, 0)),
      ],
      out_specs=[
          pl.BlockSpec((K, RB), lambda b: (0, b)),
          pl.BlockSpec((RB, 16), lambda b: (b, 0)),
      ],
      out_shape=[
          jax.ShapeDtypeStruct((K, L), jnp.int32),
          jax.ShapeDtypeStruct((L, 16), jnp.float32),
      ],
      scratch_shapes=[pltpu.VMEM((L, RB), jnp.float32),
                      pltpu.VMEM((L, RB), jnp.int32)],
      interpret=interpret,
  )(ca, ca_t, sc2, ci_f)


_GB_PER_W = (L * K) // 32
_GCHUNKS = _GB_PER_W // 128


def _sc_gather(idx3, table):
  info = plsc.get_sparse_core_info()
  nc = info.num_cores

  @functools.partial(
      pl.kernel,
      mesh=plsc.VectorSubcoreMesh(core_axis_name="c", subcore_axis_name="s"),
      compiler_params=pltpu.CompilerParams(use_tc_tiling_on_sc=False),
      out_type=jax.ShapeDtypeStruct((L * K, 16), jnp.float32),
      scratch_types=[
          pltpu.VMEM((_GCHUNKS, 128), jnp.int32),
          pltpu.VMEM((_GB_PER_W, 16), jnp.float32),
          pltpu.SemaphoreType.DMA,
      ],
  )
  def gath(idx_hbm, table_hbm, out_hbm, idx_v, rows_v, sem):
    wid = lax.axis_index("s") * nc + lax.axis_index("c")
    pltpu.sync_copy(idx_hbm.at[wid], idx_v)

    def chunk(c, carry):
      pltpu.async_copy(table_hbm.at[idx_v.at[c]],
                       rows_v.at[pl.ds(c * 128, 128), :], sem).wait()
      return carry

    lax.fori_loop(0, _GCHUNKS, chunk, 0)
    pltpu.sync_copy(rows_v, out_hbm.at[pl.ds(wid * _GB_PER_W, _GB_PER_W), :])

  return gath(idx3, table)


def _feat_body(x_ref, xn_ref, ii_ref, jj_ref, sa_ref, sb_ref, s3e_ref,
               rexp_ref, mu_ref, wbig_ref, lng_ref, lnb_ref,
               wp_ref, bp_ref, out_ref):
  dot = functools.partial(jnp.dot, precision=jax.lax.Precision.HIGHEST,
                          preferred_element_type=jnp.float32)
  dot_lo = functools.partial(jnp.dot, precision=jax.lax.Precision.DEFAULT,
                             preferred_element_type=jnp.float32)

  def split_dot(vals, sel):
    hi_p = vals.astype(jnp.bfloat16).astype(jnp.float32)
    return dot_lo(hi_p, sel) + dot_lo(vals - hi_p, sel)

  xa = dot(x_ref[...], sa_ref[...])
  xa_hi = xa.astype(jnp.bfloat16).astype(jnp.float32)
  a = (dot_lo(rexp_ref[...], xa_hi)
       + dot_lo(rexp_ref[...], xa - xa_hi))
  xn = xn_ref[...]
  xn_hi = xn.astype(jnp.bfloat16).astype(jnp.float32)
  b = (dot_lo(xn_hi, sb_ref[...])
       + dot_lo(xn - xn_hi, sb_ref[...]))
  sq = (a - b) ** 2
  x4 = split_dot(sq, s3e_ref[...])
  d4 = jnp.sqrt(x4 + 1e-06)
  rbf = jnp.exp(-(((d4 - mu_ref[...]) * 0.8) ** 2))

  d_off = jnp.clip(ii_ref[...] - jj_ref[...] + NPE, 0, 2 * NPE)
  oh = (lax.broadcasted_iota(jnp.int32, (RK, 2 * NPE + 1), 1)
        == d_off).astype(jnp.float32)
  chain = 1.0 - (a[:, 79:80] == b[:, 79:80]).astype(jnp.float32)
  feat = jnp.concatenate([rbf, oh, chain], axis=1)

  e = dot_lo(feat, wbig_ref[...])
  mu = jnp.mean(e, axis=1, keepdims=True)
  var = jnp.mean((e - mu) ** 2, axis=1, keepdims=True)
  e = (e - mu) / jnp.sqrt(var + 1e-05) * lng_ref[...] + lnb_ref[...]
  out_ref[...] = dot_lo(e, wp_ref[...]) + bp_ref[...]


def _feat_call(x, xn, ii, jj, sa, sb, s3e, rexp, mu400, wbig,
               lng, lnb, wpt, bp, interpret=False):
  full = lambda shape: pl.BlockSpec(shape, lambda b: (0, 0))
  return pl.pallas_call(
      _feat_body,
      grid=(NFEAT,),
      in_specs=[
          pl.BlockSpec((R_FEAT, 16), lambda b: (b, 0)),
          pl.BlockSpec((RK, 16), lambda b: (b, 0)),
          pl.BlockSpec((RK, 1), lambda b: (b, 0)),
          pl.BlockSpec((RK, 1), lambda b: (b, 0)),
          full((16, 80)),
          full((16, 80)),
          full((80, 400)),
          full((RK, R_FEAT)),
          full((1, 400)),
          full((434, EF)),
          full((1, EF)),
          full((1, EF)),
          full((EF, EF)),
          full((1, EF)),
      ],
      out_specs=pl.BlockSpec((RK, EF), lambda b: (b, 0)),
      out_shape=jax.ShapeDtypeStruct((L * K, EF), jnp.float32),
      interpret=interpret,
  )(x, xn, ii, jj, sa, sb, s3e, rexp, mu400, wbig, lng, lnb, wpt, bp)


def kernel(key, structure_coordinates, mask, residue_index, chain_index,
           Y, Y_t, Y_m, W_pos, W_edge, ln_g, ln_b, W_proj, b_proj):
  sc2 = structure_coordinates.reshape(L, 12)
  ci_f = chain_index.astype(jnp.float32).reshape(L, 1)
  ca = structure_coordinates[:, 1, :]
  ca_t = jnp.concatenate([ca.T, jnp.zeros((5, L), jnp.float32)], axis=0)

  eidx_t, x = _topk_call(ca, ca_t, sc2, ci_f)
  e_idx = eidx_t.T

  jflat = e_idx.reshape(L * K)
  xn = _sc_gather(jflat.reshape(32, _GCHUNKS, 128), x)

  sa, sb, s3e, rexp = _sel_consts()
  mu400 = jnp.asarray(np.tile(np.linspace(2.0, 22.0, 16,
                                          dtype=np.float32), NP_)[None, :])
  wbig = jnp.concatenate([W_edge[:, 16:].T,
                          (W_edge[:, :16] @ W_pos).T], axis=0)
  ii = jnp.repeat(jnp.arange(L, dtype=jnp.int32), K).reshape(L * K, 1)
  jj = jflat.reshape(L * K, 1).astype(jnp.int32)

  e = _feat_call(x, xn, ii, jj,
                 jnp.asarray(sa), jnp.asarray(sb), jnp.asarray(s3e),
                 jnp.asarray(rexp), mu400, wbig,
                 ln_g.reshape(1, EF), ln_b.reshape(1, EF), W_proj.T,
                 b_proj.reshape(1, EF))
  return e.reshape(L, K, EF), e_idx


# speedup vs baseline: 3.6114x; 1.0498x over previous
import functools

import numpy as np
import jax
import jax.numpy as jnp
from jax import lax
from jax.experimental import pallas as pl
from jax.experimental.pallas import tpu as pltpu
from jax.experimental.pallas import tpu_sc as plsc

L = 2048
K = 48
NPE = 16
EF = 128

RB = 128
NBLK = L // RB
CH = 128
NCH = L // CH

R_FEAT = 32
RK = R_FEAT * K
NFEAT = L // R_FEAT

_BIGF = float(np.inf)
_BIGI = 2**30

_PAIRS = [(1, 1), (0, 0), (2, 2), (3, 3), (4, 4), (1, 0), (1, 2), (1, 3),
          (1, 4), (0, 2), (0, 3), (0, 4), (4, 2), (4, 3), (3, 2), (0, 1),
          (2, 1), (3, 1), (4, 1), (2, 0), (3, 0), (4, 0), (2, 4), (3, 4),
          (2, 3)]
NP_ = len(_PAIRS)


def _sel_consts():
  sa = np.zeros((16, 80), np.float32)
  sb = np.zeros((16, 80), np.float32)
  for p, (a, b) in enumerate(_PAIRS):
    for c in range(3):
      sa[3 * a + c, 3 * p + c] = 1.0
      sb[3 * b + c, 3 * p + c] = 1.0
  sa[15, 79] = 1.0
  sb[15, 79] = 1.0
  s3e = np.zeros((80, 400), np.float32)
  for p in range(NP_):
    for c in range(3):
      s3e[3 * p + c, 16 * p:16 * p + 16] = 1.0
  r_exp = np.zeros((RK, R_FEAT), np.float32)
  for r in range(RK):
    r_exp[r, r // K] = 1.0
  return sa, sb, s3e, r_exp


def _topk_body(ca_ref, cat_ref, sc_ref, ci_ref, eidx_ref, x_ref, d_ref,
               io_ref):
  s = sc_ref[...]
  n = s[:, 0:3]
  ca_r = s[:, 3:6]
  c_r = s[:, 6:9]
  bv = ca_r - n
  cv = c_r - ca_r
  a0 = bv[:, 1:2] * cv[:, 2:3] - bv[:, 2:3] * cv[:, 1:2]
  a1 = bv[:, 2:3] * cv[:, 0:1] - bv[:, 0:1] * cv[:, 2:3]
  a2 = bv[:, 0:1] * cv[:, 1:2] - bv[:, 1:2] * cv[:, 0:1]
  av = jnp.concatenate([a0, a1, a2], axis=1)
  cb = -0.58273431 * av + 0.56802827 * bv - 0.54067466 * cv + ca_r
  x_ref[...] = jnp.concatenate([s, cb, ci_ref[...]], axis=1)

  for ch in range(NCH):
    col = ca_ref[pl.ds(ch * CH, CH), :]
    dx = col[:, 0:1] - cat_ref[0:1, :]
    dy = col[:, 1:2] - cat_ref[1:2, :]
    dz = col[:, 2:3] - cat_ref[2:3, :]
    d_ref[pl.ds(ch * CH, CH), :] = jnp.sqrt(dx * dx + dy * dy + dz * dz
                                            + 1e-06)
    io_ref[pl.ds(ch * CH, CH), :] = (
        lax.broadcasted_iota(jnp.int32, (CH, RB), 0) + ch * CH)

  def step(t, li):

    def scan_chunk(ch, mc):
      rv, ri = mc
      io = io_ref[pl.ds(ch * CH, CH), :]
      v = jnp.where(io == li, _BIGF, d_ref[pl.ds(ch * CH, CH), :])
      d_ref[pl.ds(ch * CH, CH), :] = v
      take = v < rv
      return jnp.where(take, v, rv), jnp.where(take, io, ri)

    v0 = jnp.full((CH, RB), _BIGF, jnp.float32)
    i0 = jnp.full((CH, RB), _BIGI, jnp.int32)
    rv, ri = lax.fori_loop(0, NCH, scan_chunk, (v0, i0), unroll=4)
    m = jnp.min(rv, axis=0, keepdims=True)
    mi = jnp.min(jnp.where(rv == m, ri, _BIGI), axis=0, keepdims=True)
    eidx_ref[pl.ds(t, 1), :] = mi
    return mi

  lax.fori_loop(0, K, step, jnp.full((1, RB), -1, jnp.int32))


def _topk_call(ca, ca_t, sc2, ci_f, interpret=False):
  return pl.pallas_call(
      _topk_body,
      grid=(NBLK,),
      in_specs=[
          pl.BlockSpec((L, 3), lambda b: (0, 0)),
          pl.BlockSpec((8, RB), lambda b: (0, b)),
          pl.BlockSpec((RB, 12), lambda b: (b, 0)),
          pl.BlockSpec((RB, 1), lambda b: (b, 0)),
      ],
      out_specs=[
          pl.BlockSpec((K, RB), lambda b: (0, b)),
          pl.BlockSpec((RB, 16), lambda b: (b, 0)),
      ],
      out_shape=[
          jax.ShapeDtypeStruct((K, L), jnp.int32),
          jax.ShapeDtypeStruct((L, 16), jnp.float32),
      ],
      scratch_shapes=[pltpu.VMEM((L, RB), jnp.float32),
                      pltpu.VMEM((L, RB), jnp.int32)],
      interpret=interpret,
  )(ca, ca_t, sc2, ci_f)


_GB_PER_W = (L * K) // 32
_GCHUNKS = _GB_PER_W // 128


def _sc_gather(idx3, table):
  info = plsc.get_sparse_core_info()
  nc = info.num_cores

  @functools.partial(
      pl.kernel,
      mesh=plsc.VectorSubcoreMesh(core_axis_name="c", subcore_axis_name="s"),
      compiler_params=pltpu.CompilerParams(use_tc_tiling_on_sc=False),
      out_type=jax.ShapeDtypeStruct((L * K, 16), jnp.float32),
      scratch_types=[
          pltpu.VMEM((_GCHUNKS, 128), jnp.int32),
          pltpu.VMEM((_GB_PER_W, 16), jnp.float32),
          pltpu.SemaphoreType.DMA,
      ],
  )
  def gath(idx_hbm, table_hbm, out_hbm, idx_v, rows_v, sem):
    wid = lax.axis_index("s") * nc + lax.axis_index("c")
    pltpu.sync_copy(idx_hbm.at[wid], idx_v)

    def chunk(c, carry):
      pltpu.async_copy(table_hbm.at[idx_v.at[c]],
                       rows_v.at[pl.ds(c * 128, 128), :], sem).wait()
      return carry

    lax.fori_loop(0, _GCHUNKS, chunk, 0)
    pltpu.sync_copy(rows_v, out_hbm.at[pl.ds(wid * _GB_PER_W, _GB_PER_W), :])

  return gath(idx3, table)


def _feat_body(x_ref, xn_ref, ii_ref, jj_ref, sa_ref, sb_ref, s3e_ref,
               rexp_ref, mu_ref, wbig_ref, lng_ref, lnb_ref,
               wp_ref, bp_ref, out_ref):
  dot = functools.partial(jnp.dot, precision=jax.lax.Precision.HIGHEST,
                          preferred_element_type=jnp.float32)
  dot_lo = functools.partial(jnp.dot, precision=jax.lax.Precision.DEFAULT,
                             preferred_element_type=jnp.float32)

  def split_dot(vals, sel):
    hi_p = vals.astype(jnp.bfloat16).astype(jnp.float32)
    return dot_lo(hi_p, sel) + dot_lo(vals - hi_p, sel)

  xa = dot(x_ref[...], sa_ref[...])
  xa_hi = xa.astype(jnp.bfloat16).astype(jnp.float32)
  a = (dot_lo(rexp_ref[...], xa_hi)
       + dot_lo(rexp_ref[...], xa - xa_hi))
  xn = xn_ref[...]
  xn_hi = xn.astype(jnp.bfloat16).astype(jnp.float32)
  b = (dot_lo(xn_hi, sb_ref[...])
       + dot_lo(xn - xn_hi, sb_ref[...]))
  sq = (a - b) ** 2
  x4 = split_dot(sq, s3e_ref[...])
  d4 = jnp.sqrt(x4 + 1e-06)
  rbf = jnp.exp(-(((d4 - mu_ref[...]) * 0.8) ** 2))

  d_off = jnp.clip(ii_ref[...] - jj_ref[...] + NPE, 0, 2 * NPE)
  oh = (lax.broadcasted_iota(jnp.int32, (RK, 2 * NPE + 1), 1)
        == d_off).astype(jnp.float32)
  chain = 1.0 - (a[:, 79:80] == b[:, 79:80]).astype(jnp.float32)
  feat = jnp.concatenate([rbf, oh, chain], axis=1)

  e = dot_lo(feat, wbig_ref[...])
  mu = jnp.mean(e, axis=1, keepdims=True)
  var = jnp.mean((e - mu) ** 2, axis=1, keepdims=True)
  e = (e - mu) / jnp.sqrt(var + 1e-05) * lng_ref[...] + lnb_ref[...]
  out_ref[...] = dot_lo(e, wp_ref[...]) + bp_ref[...]


def _feat_call(x, xn, ii, jj, sa, sb, s3e, rexp, mu400, wbig,
               lng, lnb, wpt, bp, interpret=False):
  full = lambda shape: pl.BlockSpec(shape, lambda b: (0, 0))
  return pl.pallas_call(
      _feat_body,
      grid=(NFEAT,),
      in_specs=[
          pl.BlockSpec((R_FEAT, 16), lambda b: (b, 0)),
          pl.BlockSpec((RK, 16), lambda b: (b, 0)),
          pl.BlockSpec((RK, 1), lambda b: (b, 0)),
          pl.BlockSpec((RK, 1), lambda b: (b, 0)),
          full((16, 80)),
          full((16, 80)),
          full((80, 400)),
          full((RK, R_FEAT)),
          full((1, 400)),
          full((434, EF)),
          full((1, EF)),
          full((1, EF)),
          full((EF, EF)),
          full((1, EF)),
      ],
      out_specs=pl.BlockSpec((RK, EF), lambda b: (b, 0)),
      out_shape=jax.ShapeDtypeStruct((L * K, EF), jnp.float32),
      interpret=interpret,
  )(x, xn, ii, jj, sa, sb, s3e, rexp, mu400, wbig, lng, lnb, wpt, bp)


def kernel(key, structure_coordinates, mask, residue_index, chain_index,
           Y, Y_t, Y_m, W_pos, W_edge, ln_g, ln_b, W_proj, b_proj):
  sc2 = structure_coordinates.reshape(L, 12)
  ci_f = chain_index.astype(jnp.float32).reshape(L, 1)
  ca = structure_coordinates[:, 1, :]
  ca_t = jnp.concatenate([ca.T, jnp.zeros((5, L), jnp.float32)], axis=0)

  eidx_t, x = _topk_call(ca, ca_t, sc2, ci_f)
  e_idx = eidx_t.T

  jflat = e_idx.reshape(L * K)
  xn = _sc_gather(jflat.reshape(32, _GCHUNKS, 128), x)

  sa, sb, s3e, rexp = _sel_consts()
  mu400 = jnp.asarray(np.tile(np.linspace(2.0, 22.0, 16,
                                          dtype=np.float32), NP_)[None, :])
  wbig = jnp.concatenate([W_edge[:, 16:].T,
                          (W_edge[:, :16] @ W_pos).T], axis=0)
  ii = jnp.repeat(jnp.arange(L, dtype=jnp.int32), K).reshape(L * K, 1)
  jj = jflat.reshape(L * K, 1).astype(jnp.int32)

  e = _feat_call(x, xn, ii, jj,
                 jnp.asarray(sa), jnp.asarray(sb), jnp.asarray(s3e),
                 jnp.asarray(rexp), mu400, wbig,
                 ln_g.reshape(1, EF), ln_b.reshape(1, EF), W_proj.T,
                 b_proj.reshape(1, EF))
  return e.reshape(L, K, EF), e_idx


# speedup vs baseline: 3.8480x; 1.0655x over previous
import functools

import numpy as np
import jax
import jax.numpy as jnp
from jax import lax
from jax.experimental import pallas as pl
from jax.experimental.pallas import tpu as pltpu
from jax.experimental.pallas import tpu_sc as plsc

L = 2048
K = 48
NPE = 16
EF = 128

RB = 128
NBLK = L // RB
CH = 128
NCH = L // CH

R_FEAT = 64
RK = R_FEAT * K
NFEAT = L // R_FEAT

_BIGF = float(np.inf)
_BIGI = 2**30

_PAIRS = [(1, 1), (0, 0), (2, 2), (3, 3), (4, 4), (1, 0), (1, 2), (1, 3),
          (1, 4), (0, 2), (0, 3), (0, 4), (4, 2), (4, 3), (3, 2), (0, 1),
          (2, 1), (3, 1), (4, 1), (2, 0), (3, 0), (4, 0), (2, 4), (3, 4),
          (2, 3)]
NP_ = len(_PAIRS)


def _sel_consts():
  sa = np.zeros((16, 80), np.float32)
  sb = np.zeros((16, 80), np.float32)
  for p, (a, b) in enumerate(_PAIRS):
    for c in range(3):
      sa[3 * a + c, 3 * p + c] = 1.0
      sb[3 * b + c, 3 * p + c] = 1.0
  sa[15, 79] = 1.0
  sb[15, 79] = 1.0
  s3e = np.zeros((80, 400), np.float32)
  for p in range(NP_):
    for c in range(3):
      s3e[3 * p + c, 16 * p:16 * p + 16] = 1.0
  r_exp = np.zeros((RK, R_FEAT), np.float32)
  for r in range(RK):
    r_exp[r, r // K] = 1.0
  return sa, sb, s3e, r_exp


def _topk_body(ca_ref, cat_ref, sc_ref, ci_ref, eidx_ref, x_ref, d_ref,
               io_ref):
  s = sc_ref[...]
  n = s[:, 0:3]
  ca_r = s[:, 3:6]
  c_r = s[:, 6:9]
  bv = ca_r - n
  cv = c_r - ca_r
  a0 = bv[:, 1:2] * cv[:, 2:3] - bv[:, 2:3] * cv[:, 1:2]
  a1 = bv[:, 2:3] * cv[:, 0:1] - bv[:, 0:1] * cv[:, 2:3]
  a2 = bv[:, 0:1] * cv[:, 1:2] - bv[:, 1:2] * cv[:, 0:1]
  av = jnp.concatenate([a0, a1, a2], axis=1)
  cb = -0.58273431 * av + 0.56802827 * bv - 0.54067466 * cv + ca_r
  x_ref[...] = jnp.concatenate([s, cb, ci_ref[...]], axis=1)

  for ch in range(NCH):
    col = ca_ref[pl.ds(ch * CH, CH), :]
    dx = col[:, 0:1] - cat_ref[0:1, :]
    dy = col[:, 1:2] - cat_ref[1:2, :]
    dz = col[:, 2:3] - cat_ref[2:3, :]
    d_ref[pl.ds(ch * CH, CH), :] = jnp.sqrt(dx * dx + dy * dy + dz * dz
                                            + 1e-06)
    io_ref[pl.ds(ch * CH, CH), :] = (
        lax.broadcasted_iota(jnp.int32, (CH, RB), 0) + ch * CH)

  def step(t, li):

    def scan_chunk(ch, mc):
      rv, ri = mc
      io = io_ref[pl.ds(ch * CH, CH), :]
      v = jnp.where(io == li, _BIGF, d_ref[pl.ds(ch * CH, CH), :])
      d_ref[pl.ds(ch * CH, CH), :] = v
      take = v < rv
      return jnp.where(take, v, rv), jnp.where(take, io, ri)

    v0 = jnp.full((CH, RB), _BIGF, jnp.float32)
    i0 = jnp.full((CH, RB), _BIGI, jnp.int32)
    rv, ri = lax.fori_loop(0, NCH, scan_chunk, (v0, i0), unroll=8)
    m = jnp.min(rv, axis=0, keepdims=True)
    mi = jnp.min(jnp.where(rv == m, ri, _BIGI), axis=0, keepdims=True)
    eidx_ref[pl.ds(t, 1), :] = mi
    return mi

  lax.fori_loop(0, K, step, jnp.full((1, RB), -1, jnp.int32))


def _topk_call(ca, ca_t, sc2, ci_f, interpret=False):
  return pl.pallas_call(
      _topk_body,
      grid=(NBLK,),
      in_specs=[
          pl.BlockSpec((L, 3), lambda b: (0, 0)),
          pl.BlockSpec((8, RB), lambda b: (0, b)),
          pl.BlockSpec((RB, 12), lambda b: (b, 0)),
          pl.BlockSpec((RB, 1), lambda b: (b, 0)),
      ],
      out_specs=[
          pl.BlockSpec((K, RB), lambda b: (0, b)),
          pl.BlockSpec((RB, 16), lambda b: (b, 0)),
      ],
      out_shape=[
          jax.ShapeDtypeStruct((K, L), jnp.int32),
          jax.ShapeDtypeStruct((L, 16), jnp.float32),
      ],
      scratch_shapes=[pltpu.VMEM((L, RB), jnp.float32),
                      pltpu.VMEM((L, RB), jnp.int32)],
      interpret=interpret,
  )(ca, ca_t, sc2, ci_f)


_GB_PER_W = (L * K) // 32
_GCHUNKS = _GB_PER_W // 128


def _sc_gather(idx3, table):
  info = plsc.get_sparse_core_info()
  nc = info.num_cores

  @functools.partial(
      pl.kernel,
      mesh=plsc.VectorSubcoreMesh(core_axis_name="c", subcore_axis_name="s"),
      compiler_params=pltpu.CompilerParams(use_tc_tiling_on_sc=False),
      out_type=jax.ShapeDtypeStruct((L * K, 16), jnp.float32),
      scratch_types=[
          pltpu.VMEM((_GCHUNKS, 128), jnp.int32),
          pltpu.VMEM((_GB_PER_W, 16), jnp.float32),
          pltpu.SemaphoreType.DMA,
      ],
  )
  def gath(idx_hbm, table_hbm, out_hbm, idx_v, rows_v, sem):
    wid = lax.axis_index("s") * nc + lax.axis_index("c")
    pltpu.sync_copy(idx_hbm.at[wid], idx_v)

    def chunk(c, carry):
      pltpu.async_copy(table_hbm.at[idx_v.at[c]],
                       rows_v.at[pl.ds(c * 128, 128), :], sem).wait()
      return carry

    lax.fori_loop(0, _GCHUNKS, chunk, 0)
    pltpu.sync_copy(rows_v, out_hbm.at[pl.ds(wid * _GB_PER_W, _GB_PER_W), :])

  return gath(idx3, table)


def _feat_body(x_ref, xn_ref, ii_ref, jj_ref, sa_ref, sb_ref, s3e_ref,
               rexp_ref, mu_ref, wbig_ref, lng_ref, lnb_ref,
               wp_ref, bp_ref, out_ref):
  dot = functools.partial(jnp.dot, precision=jax.lax.Precision.HIGHEST,
                          preferred_element_type=jnp.float32)
  dot_lo = functools.partial(jnp.dot, precision=jax.lax.Precision.DEFAULT,
                             preferred_element_type=jnp.float32)

  def split_dot(vals, sel):
    hi_p = vals.astype(jnp.bfloat16).astype(jnp.float32)
    return dot_lo(hi_p, sel) + dot_lo(vals - hi_p, sel)

  xa = dot(x_ref[...], sa_ref[...])
  xa_hi = xa.astype(jnp.bfloat16).astype(jnp.float32)
  a = (dot_lo(rexp_ref[...], xa_hi)
       + dot_lo(rexp_ref[...], xa - xa_hi))
  xn = xn_ref[...]
  xn_hi = xn.astype(jnp.bfloat16).astype(jnp.float32)
  b = (dot_lo(xn_hi, sb_ref[...])
       + dot_lo(xn - xn_hi, sb_ref[...]))
  sq = (a - b) ** 2
  x4 = split_dot(sq, s3e_ref[...])
  d4 = jnp.sqrt(x4 + 1e-06)
  rbf = jnp.exp(-(((d4 - mu_ref[...]) * 0.8) ** 2))

  d_off = jnp.clip(ii_ref[...] - jj_ref[...] + NPE, 0, 2 * NPE)
  oh = (lax.broadcasted_iota(jnp.int32, (RK, 2 * NPE + 1), 1)
        == d_off).astype(jnp.float32)
  chain = 1.0 - (a[:, 79:80] == b[:, 79:80]).astype(jnp.float32)
  feat = jnp.concatenate([rbf, oh, chain], axis=1)

  e = dot_lo(feat, wbig_ref[...])
  mu = jnp.mean(e, axis=1, keepdims=True)
  var = jnp.mean((e - mu) ** 2, axis=1, keepdims=True)
  e = (e - mu) / jnp.sqrt(var + 1e-05) * lng_ref[...] + lnb_ref[...]
  out_ref[...] = dot_lo(e, wp_ref[...]) + bp_ref[...]


def _feat_call(x, xn, ii, jj, sa, sb, s3e, rexp, mu400, wbig,
               lng, lnb, wpt, bp, interpret=False):
  full = lambda shape: pl.BlockSpec(shape, lambda b: (0, 0))
  return pl.pallas_call(
      _feat_body,
      grid=(NFEAT,),
      in_specs=[
          pl.BlockSpec((R_FEAT, 16), lambda b: (b, 0)),
          pl.BlockSpec((RK, 16), lambda b: (b, 0)),
          pl.BlockSpec((RK, 1), lambda b: (b, 0)),
          pl.BlockSpec((RK, 1), lambda b: (b, 0)),
          full((16, 80)),
          full((16, 80)),
          full((80, 400)),
          full((RK, R_FEAT)),
          full((1, 400)),
          full((434, EF)),
          full((1, EF)),
          full((1, EF)),
          full((EF, EF)),
          full((1, EF)),
      ],
      out_specs=pl.BlockSpec((RK, EF), lambda b: (b, 0)),
      out_shape=jax.ShapeDtypeStruct((L * K, EF), jnp.float32),
      interpret=interpret,
  )(x, xn, ii, jj, sa, sb, s3e, rexp, mu400, wbig, lng, lnb, wpt, bp)


def kernel(key, structure_coordinates, mask, residue_index, chain_index,
           Y, Y_t, Y_m, W_pos, W_edge, ln_g, ln_b, W_proj, b_proj):
  sc2 = structure_coordinates.reshape(L, 12)
  ci_f = chain_index.astype(jnp.float32).reshape(L, 1)
  ca = structure_coordinates[:, 1, :]
  ca_t = jnp.concatenate([ca.T, jnp.zeros((5, L), jnp.float32)], axis=0)

  eidx_t, x = _topk_call(ca, ca_t, sc2, ci_f)
  e_idx = eidx_t.T

  jflat = e_idx.reshape(L * K)
  xn = _sc_gather(jflat.reshape(32, _GCHUNKS, 128), x)

  sa, sb, s3e, rexp = _sel_consts()
  mu400 = jnp.asarray(np.tile(np.linspace(2.0, 22.0, 16,
                                          dtype=np.float32), NP_)[None, :])
  wbig = jnp.concatenate([W_edge[:, 16:].T,
                          (W_edge[:, :16] @ W_pos).T], axis=0)
  ii = jnp.repeat(jnp.arange(L, dtype=jnp.int32), K).reshape(L * K, 1)
  jj = jflat.reshape(L * K, 1).astype(jnp.int32)

  e = _feat_call(x, xn, ii, jj,
                 jnp.asarray(sa), jnp.asarray(sb), jnp.asarray(s3e),
                 jnp.asarray(rexp), mu400, wbig,
                 ln_g.reshape(1, EF), ln_b.reshape(1, EF), W_proj.T,
                 b_proj.reshape(1, EF))
  return e.reshape(L, K, EF), e_idx


# speedup vs baseline: 3.9035x; 1.0144x over previous
import functools

import numpy as np
import jax
import jax.numpy as jnp
from jax import lax
from jax.experimental import pallas as pl
from jax.experimental.pallas import tpu as pltpu
from jax.experimental.pallas import tpu_sc as plsc

L = 2048
K = 48
NPE = 16
EF = 128

RB = 128
NBLK = L // RB
CH = 128
NCH = L // CH

R_FEAT = 64
RK = R_FEAT * K
NFEAT = L // R_FEAT

_BIGF = float(np.inf)
_BIGI = 2**30

_PAIRS = [(1, 1), (0, 0), (2, 2), (3, 3), (4, 4), (1, 0), (1, 2), (1, 3),
          (1, 4), (0, 2), (0, 3), (0, 4), (4, 2), (4, 3), (3, 2), (0, 1),
          (2, 1), (3, 1), (4, 1), (2, 0), (3, 0), (4, 0), (2, 4), (3, 4),
          (2, 3)]
NP_ = len(_PAIRS)


def _sel_consts():
  sa = np.zeros((16, 80), np.float32)
  sb = np.zeros((16, 80), np.float32)
  for p, (a, b) in enumerate(_PAIRS):
    for c in range(3):
      sa[3 * a + c, 3 * p + c] = 1.0
      sb[3 * b + c, 3 * p + c] = 1.0
  sa[15, 79] = 1.0
  sb[15, 79] = 1.0
  s3e = np.zeros((80, 400), np.float32)
  for p in range(NP_):
    for c in range(3):
      s3e[3 * p + c, 16 * p:16 * p + 16] = 1.0
  r_exp = np.zeros((RK, R_FEAT), np.float32)
  for r in range(RK):
    r_exp[r, r // K] = 1.0
  return sa, sb, s3e, r_exp


def _topk_body(ca_ref, cat_ref, sc_ref, ci_ref, eidx_ref, x_ref, d_ref,
               io_ref):
  s = sc_ref[...]
  n = s[:, 0:3]
  ca_r = s[:, 3:6]
  c_r = s[:, 6:9]
  bv = ca_r - n
  cv = c_r - ca_r
  a0 = bv[:, 1:2] * cv[:, 2:3] - bv[:, 2:3] * cv[:, 1:2]
  a1 = bv[:, 2:3] * cv[:, 0:1] - bv[:, 0:1] * cv[:, 2:3]
  a2 = bv[:, 0:1] * cv[:, 1:2] - bv[:, 1:2] * cv[:, 0:1]
  av = jnp.concatenate([a0, a1, a2], axis=1)
  cb = -0.58273431 * av + 0.56802827 * bv - 0.54067466 * cv + ca_r
  x_ref[...] = jnp.concatenate([s, cb, ci_ref[...]], axis=1)

  for ch in range(NCH):
    col = ca_ref[pl.ds(ch * CH, CH), :]
    dx = col[:, 0:1] - cat_ref[0:1, :]
    dy = col[:, 1:2] - cat_ref[1:2, :]
    dz = col[:, 2:3] - cat_ref[2:3, :]
    d_ref[pl.ds(ch * CH, CH), :] = jnp.sqrt(dx * dx + dy * dy + dz * dz
                                            + 1e-06)
    io_ref[pl.ds(ch * CH, CH), :] = (
        lax.broadcasted_iota(jnp.int32, (CH, RB), 0) + ch * CH)

  def step(t, li):

    def scan_chunk(ch, mc):
      rv, ri = mc
      io = io_ref[pl.ds(ch * CH, CH), :]
      v = jnp.where(io == li, _BIGF, d_ref[pl.ds(ch * CH, CH), :])
      d_ref[pl.ds(ch * CH, CH), :] = v
      take = v < rv
      return jnp.where(take, v, rv), jnp.where(take, io, ri)

    v0 = jnp.full((CH, RB), _BIGF, jnp.float32)
    i0 = jnp.full((CH, RB), _BIGI, jnp.int32)
    rv, ri = lax.fori_loop(0, NCH, scan_chunk, (v0, i0), unroll=True)
    m = jnp.min(rv, axis=0, keepdims=True)
    mi = jnp.min(jnp.where(rv == m, ri, _BIGI), axis=0, keepdims=True)
    eidx_ref[pl.ds(t, 1), :] = mi
    return mi

  lax.fori_loop(0, K, step, jnp.full((1, RB), -1, jnp.int32), unroll=2)


def _topk_call(ca, ca_t, sc2, ci_f, interpret=False):
  return pl.pallas_call(
      _topk_body,
      grid=(NBLK,),
      in_specs=[
          pl.BlockSpec((L, 3), lambda b: (0, 0)),
          pl.BlockSpec((8, RB), lambda b: (0, b)),
          pl.BlockSpec((RB, 12), lambda b: (b, 0)),
          pl.BlockSpec((RB, 1), lambda b: (b, 0)),
      ],
      out_specs=[
          pl.BlockSpec((K, RB), lambda b: (0, b)),
          pl.BlockSpec((RB, 16), lambda b: (b, 0)),
      ],
      out_shape=[
          jax.ShapeDtypeStruct((K, L), jnp.int32),
          jax.ShapeDtypeStruct((L, 16), jnp.float32),
      ],
      scratch_shapes=[pltpu.VMEM((L, RB), jnp.float32),
                      pltpu.VMEM((L, RB), jnp.int32)],
      interpret=interpret,
  )(ca, ca_t, sc2, ci_f)


_GB_PER_W = (L * K) // 32
_GCHUNKS = _GB_PER_W // 128


def _sc_gather(idx3, table):
  info = plsc.get_sparse_core_info()
  nc = info.num_cores

  @functools.partial(
      pl.kernel,
      mesh=plsc.VectorSubcoreMesh(core_axis_name="c", subcore_axis_name="s"),
      compiler_params=pltpu.CompilerParams(use_tc_tiling_on_sc=False),
      out_type=jax.ShapeDtypeStruct((L * K, 16), jnp.float32),
      scratch_types=[
          pltpu.VMEM((_GCHUNKS, 128), jnp.int32),
          pltpu.VMEM((_GB_PER_W, 16), jnp.float32),
          pltpu.SemaphoreType.DMA,
      ],
  )
  def gath(idx_hbm, table_hbm, out_hbm, idx_v, rows_v, sem):
    wid = lax.axis_index("s") * nc + lax.axis_index("c")
    pltpu.sync_copy(idx_hbm.at[wid], idx_v)

    def chunk(c, carry):
      pltpu.async_copy(table_hbm.at[idx_v.at[c]],
                       rows_v.at[pl.ds(c * 128, 128), :], sem).wait()
      return carry

    lax.fori_loop(0, _GCHUNKS, chunk, 0)
    pltpu.sync_copy(rows_v, out_hbm.at[pl.ds(wid * _GB_PER_W, _GB_PER_W), :])

  return gath(idx3, table)


def _feat_body(x_ref, xn_ref, ii_ref, jj_ref, sa_ref, sb_ref, s3e_ref,
               rexp_ref, mu_ref, wbig_ref, lng_ref, lnb_ref,
               wp_ref, bp_ref, out_ref):
  dot = functools.partial(jnp.dot, precision=jax.lax.Precision.HIGHEST,
                          preferred_element_type=jnp.float32)
  dot_lo = functools.partial(jnp.dot, precision=jax.lax.Precision.DEFAULT,
                             preferred_element_type=jnp.float32)

  def split_dot(vals, sel):
    hi_p = vals.astype(jnp.bfloat16).astype(jnp.float32)
    return dot_lo(hi_p, sel) + dot_lo(vals - hi_p, sel)

  xa = dot(x_ref[...], sa_ref[...])
  xa_hi = xa.astype(jnp.bfloat16).astype(jnp.float32)
  a = (dot_lo(rexp_ref[...], xa_hi)
       + dot_lo(rexp_ref[...], xa - xa_hi))
  xn = xn_ref[...]
  xn_hi = xn.astype(jnp.bfloat16).astype(jnp.float32)
  b = (dot_lo(xn_hi, sb_ref[...])
       + dot_lo(xn - xn_hi, sb_ref[...]))
  sq = (a - b) ** 2
  x4 = split_dot(sq, s3e_ref[...])
  d4 = jnp.sqrt(x4 + 1e-06)
  rbf = jnp.exp(-(((d4 - mu_ref[...]) * 0.8) ** 2))

  d_off = jnp.clip(ii_ref[...] - jj_ref[...] + NPE, 0, 2 * NPE)
  oh = (lax.broadcasted_iota(jnp.int32, (RK, 2 * NPE + 1), 1)
        == d_off).astype(jnp.float32)
  chain = 1.0 - (a[:, 79:80] == b[:, 79:80]).astype(jnp.float32)
  feat = jnp.concatenate([rbf, oh, chain], axis=1)

  e = dot_lo(feat, wbig_ref[...])
  mu = jnp.mean(e, axis=1, keepdims=True)
  var = jnp.mean((e - mu) ** 2, axis=1, keepdims=True)
  e = (e - mu) / jnp.sqrt(var + 1e-05) * lng_ref[...] + lnb_ref[...]
  out_ref[...] = dot_lo(e, wp_ref[...]) + bp_ref[...]


def _feat_call(x, xn, ii, jj, sa, sb, s3e, rexp, mu400, wbig,
               lng, lnb, wpt, bp, interpret=False):
  full = lambda shape: pl.BlockSpec(shape, lambda b: (0, 0))
  return pl.pallas_call(
      _feat_body,
      grid=(NFEAT,),
      in_specs=[
          pl.BlockSpec((R_FEAT, 16), lambda b: (b, 0)),
          pl.BlockSpec((RK, 16), lambda b: (b, 0)),
          pl.BlockSpec((RK, 1), lambda b: (b, 0)),
          pl.BlockSpec((RK, 1), lambda b: (b, 0)),
          full((16, 80)),
          full((16, 80)),
          full((80, 400)),
          full((RK, R_FEAT)),
          full((1, 400)),
          full((434, EF)),
          full((1, EF)),
          full((1, EF)),
          full((EF, EF)),
          full((1, EF)),
      ],
      out_specs=pl.BlockSpec((RK, EF), lambda b: (b, 0)),
      out_shape=jax.ShapeDtypeStruct((L * K, EF), jnp.float32),
      interpret=interpret,
  )(x, xn, ii, jj, sa, sb, s3e, rexp, mu400, wbig, lng, lnb, wpt, bp)


def kernel(key, structure_coordinates, mask, residue_index, chain_index,
           Y, Y_t, Y_m, W_pos, W_edge, ln_g, ln_b, W_proj, b_proj):
  sc2 = structure_coordinates.reshape(L, 12)
  ci_f = chain_index.astype(jnp.float32).reshape(L, 1)
  ca = structure_coordinates[:, 1, :]
  ca_t = jnp.concatenate([ca.T, jnp.zeros((5, L), jnp.float32)], axis=0)

  eidx_t, x = _topk_call(ca, ca_t, sc2, ci_f)
  e_idx = eidx_t.T

  jflat = e_idx.reshape(L * K)
  xn = _sc_gather(jflat.reshape(32, _GCHUNKS, 128), x)

  sa, sb, s3e, rexp = _sel_consts()
  mu400 = jnp.asarray(np.tile(np.linspace(2.0, 22.0, 16,
                                          dtype=np.float32), NP_)[None, :])
  wbig = jnp.concatenate([W_edge[:, 16:].T,
                          (W_edge[:, :16] @ W_pos).T], axis=0)
  ii = jnp.repeat(jnp.arange(L, dtype=jnp.int32), K).reshape(L * K, 1)
  jj = jflat.reshape(L * K, 1).astype(jnp.int32)

  e = _feat_call(x, xn, ii, jj,
                 jnp.asarray(sa), jnp.asarray(sb), jnp.asarray(s3e),
                 jnp.asarray(rexp), mu400, wbig,
                 ln_g.reshape(1, EF), ln_b.reshape(1, EF), W_proj.T,
                 b_proj.reshape(1, EF))
  return e.reshape(L, K, EF), e_idx
